# baseline (speedup 1.0000x reference)
import functools

import jax
import jax.numpy as jnp
from jax import lax
from jax.experimental import pallas as pl
from jax.experimental.pallas import tpu as pltpu
from jax.experimental.pallas import tpu_sc as plsc

N = 50000
D = 128
NP = 50176
CS = 12544
ACC_R = 12672
E = 500000
EP = 507904
EB = 512
NB = EP // (16 * EB)
TS = EP // 16
FL = 64
NG = 16

_SDS = jax.ShapeDtypeStruct


def _agg_body(table, src, dst, out, acc, cnt, esrc, edst, ga, da,
              gt0, gt1, dt0, dt1, rowbuf, nbuf, cbuf, ones_f, gsem, esem):
    core = lax.axis_index("c")
    sub = lax.axis_index("s")
    lane = lax.iota(jnp.int32, 16)
    zero16 = jnp.zeros((16,), jnp.int32)
    gt = (gt0, gt1)
    dt = (dt0, dt1)

    for q in range(4):
        ones_f[pl.ds(16 * q, 16)] = jnp.ones((16,), jnp.float32)

    def mk_flush(k):
        o = 1 - k

        def f(cur):
            for q in range(FL // 16):
                gt[k][pl.ds(16 * q, 16)] = ga[pl.ds(16 * q, 16)]
                dt[k][pl.ds(16 * q, 16)] = da[pl.ds(16 * q, 16)]
            spill = cur - FL
            vg = ga[pl.ds(FL, 16)]
            vd = da[pl.ds(FL, 16)]
            mm = lane < spill
            plsc.store_compressed(ga.at[pl.ds(0, 16)], vg, mask=mm)
            plsc.store_compressed(da.at[pl.ds(0, 16)], vd, mask=mm)
            pltpu.async_copy(table.at[gt[k]], rowbuf.at[k], gsem).wait()
            pltpu.sync_copy(rowbuf.at[k], acc.at[dt[k]], add=True)
            pltpu.sync_copy(ones_f, cnt.at[dt[k]], add=True)
            return spill
        return f

    flush0 = mk_flush(0)
    flush1 = mk_flush(1)

    def drain(k):
        pltpu.make_async_copy(table.at[gt[k]], rowbuf.at[k], gsem).wait()
        pltpu.sync_copy(rowbuf.at[k], acc.at[dt[k]], add=True)
        pltpu.sync_copy(ones_f, cnt.at[dt[k]], add=True)

    def mk_vloop(b, lo):
        def vreg_body(k, carry):
            cur, slot = carry
            s = esrc[b, pl.ds(16 * k, 16)]
            d = edst[b, pl.ds(16 * k, 16)]
            loc = d - lo
            m = (loc >= 0) & (loc < CS)
            plsc.store_compressed(ga.at[pl.ds(cur, 16)], s, mask=m)
            plsc.store_compressed(da.at[pl.ds(cur, 16)], loc, mask=m)
            cur = cur + plsc.all_reduce_population_count(m)[0]

            def do_flush(args):
                cc, ss = args
                nc = lax.cond(ss == 0, flush0, flush1, cc)
                return (nc, 1 - ss)
            return lax.cond(cur >= FL, do_flush, lambda a: a, (cur, slot))
        return vreg_body

    for c in range(2):
        cid = core * 2 + c
        lo = cid * CS
        vloop0 = mk_vloop(0, lo)
        vloop1 = mk_vloop(1, lo)

        def zb(r, carry):
            for q in range(8):
                rowbuf[0, r, pl.ds(16 * q, 16)] = jnp.zeros((16,), jnp.float32)
            return carry
        lax.fori_loop(0, FL, zb, 0)
        rz = 792 * sub
        for j2 in range(12):
            pltpu.sync_copy(rowbuf.at[0], acc.at[pl.ds(rz + FL * j2, FL)])
        pltpu.sync_copy(rowbuf.at[0, pl.ds(0, 24)], acc.at[pl.ds(rz + 768, 24)])
        for j2 in range(6):
            pltpu.sync_copy(rowbuf.at[0, 0], cnt.at[pl.ds(rz + 128 * j2, 128)])
        pltpu.sync_copy(rowbuf.at[0, 0, pl.ds(0, 24)],
                        cnt.at[pl.ds(rz + 768, 24)])
        plsc.subcore_barrier()

        def batch_body(j, carry):
            eb0 = sub * TS + EB * j
            pltpu.sync_copy(src.at[pl.ds(eb0, EB)], esrc.at[0])
            pltpu.sync_copy(dst.at[pl.ds(eb0, EB)], edst.at[0])
            return lax.fori_loop(0, EB // 16, vloop0, carry)

        cursor, slot = lax.fori_loop(0, NB, batch_body,
                                     (jnp.int32(0), jnp.int32(0)))

        def pad_body(p, cur):
            pp = 16 * p + lane + cur
            dsrc = pp * 157 + sub * 16
            dloc = CS + lax.bitwise_and(pp + sub * 8, 127)
            ga[pl.ds(cur + 16 * p, 16)] = dsrc
            da[pl.ds(cur + 16 * p, 16)] = dloc
            return cur
        lax.fori_loop(0, FL // 16, pad_body, cursor)
        lax.cond(slot == 0, flush0, flush1, jnp.int32(FL))
        plsc.subcore_barrier()

        nb_base = 784 * sub

        def norm_body(b, carry):
            rb = nb_base + 16 * b
            pltpu.sync_copy(acc.at[pl.ds(rb, 16)], nbuf)
            pltpu.sync_copy(cnt.at[pl.ds(rb, 16)], cbuf)
            cv = cbuf[pl.ds(0, 16)]
            iv = 1.0 / jnp.maximum(cv, 1.0)
            for r in range(16):
                sc = iv[r]
                for q in range(8):
                    v = nbuf[r, pl.ds(16 * q, 16)]
                    nbuf[r, pl.ds(16 * q, 16)] = jnp.maximum(v * sc, 0.0)
            pltpu.sync_copy(nbuf, out.at[pl.ds(lo + rb, 16)])
            return carry
        lax.fori_loop(0, 49, norm_body, 0)
        plsc.subcore_barrier()


_agg = pl.kernel(
    _agg_body,
    out_type=_SDS((NP, D), jnp.float32),
    mesh=plsc.VectorSubcoreMesh(core_axis_name="c", subcore_axis_name="s"),
    scratch_types=[
        pltpu.VMEM_SHARED((ACC_R, D), jnp.float32),
        pltpu.VMEM_SHARED((ACC_R,), jnp.float32),
        pltpu.VMEM((2, EB), jnp.int32),
        pltpu.VMEM((2, EB), jnp.int32),
        pltpu.VMEM((2 * FL,), jnp.int32),
        pltpu.VMEM((2 * FL,), jnp.int32),
        pltpu.VMEM((FL,), jnp.int32),
        pltpu.VMEM((FL,), jnp.int32),
        pltpu.VMEM((FL,), jnp.int32),
        pltpu.VMEM((FL,), jnp.int32),
        pltpu.VMEM((2, FL, D), jnp.float32),
        pltpu.VMEM((16, D), jnp.float32),
        pltpu.VMEM((16,), jnp.float32),
        pltpu.VMEM((FL,), jnp.float32),
        pltpu.SemaphoreType.DMA,
        pltpu.SemaphoreType.DMA,
    ],
    compiler_params=pltpu.CompilerParams(needs_layout_passes=False),
    name="hgnn_mean_agg",
)

POOL_ROWS = NP // 32


def _pool_body(x, b, psum, pub, rbuf, bbuf, pbuf, tbuf):
    core = lax.axis_index("c")
    sub = lax.axis_index("s")
    lane = lax.iota(jnp.int32, 16)
    w = core * 16 + sub

    def zp(r, carry):
        for q in range(8):
            pbuf[r, pl.ds(16 * q, 16)] = jnp.zeros((16,), jnp.float32)
        return carry
    lax.fori_loop(0, 32, zp, 0)

    def bb(i, carry):
        st = w * POOL_ROWS + 112 * i
        pltpu.sync_copy(x.at[pl.ds(st, 112)], rbuf)
        pltpu.sync_copy(b.at[pl.ds(st, 112)], bbuf)

        def grp(g, c2):
            bidv = bbuf[pl.ds(16 * g, 16)]
            for r in range(16):
                rowsv = jnp.zeros((16,), jnp.int32) + bidv[r]
                for q in range(8):
                    plsc.addupdate_scatter(
                        pbuf, [rowsv, 16 * q + lane],
                        rbuf[16 * g + r, pl.ds(16 * q, 16)])
            return c2
        lax.fori_loop(0, 7, grp, 0)
        return carry
    lax.fori_loop(0, POOL_ROWS // 112, bb, 0)

    pltpu.sync_copy(pbuf, pub.at[pl.ds(32 * sub, 32)])
    plsc.subcore_barrier()

    @pl.when(sub == 0)
    def _():
        lax.fori_loop(0, 32, zp, 0)

        def mg(t, c):
            pltpu.sync_copy(pub.at[pl.ds(32 * t, 32)], tbuf)
            for r in range(32):
                for q in range(8):
                    pbuf[r, pl.ds(16 * q, 16)] = (
                        pbuf[r, pl.ds(16 * q, 16)]
                        + tbuf[r, pl.ds(16 * q, 16)])
            return c
        lax.fori_loop(0, 16, mg, 0)
        pltpu.sync_copy(pbuf.at[pl.ds(0, NG)], psum.at[core])


_pool = pl.kernel(
    _pool_body,
    out_type=_SDS((2, NG, D), jnp.float32),
    mesh=plsc.VectorSubcoreMesh(core_axis_name="c", subcore_axis_name="s"),
    scratch_types=[
        pltpu.VMEM_SHARED((512, D), jnp.float32),
        pltpu.VMEM((112, D), jnp.float32),
        pltpu.VMEM((112,), jnp.int32),
        pltpu.VMEM((32, D), jnp.float32),
        pltpu.VMEM((32, D), jnp.float32),
    ],
    compiler_params=pltpu.CompilerParams(needs_layout_passes=False),
    name="hgnn_pool",
)


def _comb_body(s_ref, b_ref, o_ref):
    s = s_ref[0] + s_ref[1]
    for g in range(NG):
        cg = jnp.sum(jnp.where(b_ref[...] == g, 1.0, 0.0))
        o_ref[g, :] = s[g, :] / jnp.maximum(cg, 1.0)


def _combine(psum, b2d):
    return pl.pallas_call(
        _comb_body,
        out_shape=_SDS((NG, D), jnp.float32),
    )(psum, b2d)


def kernel(x_node, x_link, edge_index_nl, edge_index_ln, batch):
    npad = EP - E
    pad_src = ((jnp.arange(npad, dtype=jnp.int32) * 7919) % N).astype(jnp.int32)
    pad_dst = (N + jnp.arange(npad, dtype=jnp.int32) % (NP - N)).astype(jnp.int32)
    nl_s = jnp.concatenate([edge_index_nl[0], pad_src])
    nl_d = jnp.concatenate([edge_index_nl[1], pad_dst])
    ln_s = jnp.concatenate([edge_index_ln[0], pad_src])
    ln_d = jnp.concatenate([edge_index_ln[1], pad_dst])
    xp = jnp.concatenate([x_node, jnp.zeros((NP - N, D), jnp.float32)])
    link1 = _agg(xp, nl_s, nl_d)
    node2 = _agg(link1, ln_s, ln_d)
    link3 = _agg(node2, nl_s, nl_d)
    bpad = (NG + jnp.arange(NP - N, dtype=jnp.int32) % NG).astype(jnp.int32)
    bp = jnp.concatenate([batch, bpad])
    psum = _pool(link3, bp)
    return _combine(psum, bp.reshape(NP // D, D))
